# Initial kernel scaffold; baseline (speedup 1.0000x reference)
import jax
import jax.numpy as jnp
from jax.experimental import pallas as pl


def kernel(x, edge_index, pW1, pb1, pg1, pbe1, pW2, pb2, uW1, ub1, ug1, ube1, uW2, ub2, rW1, rb1, rg1, rbe1, rW2, rb2):
    raise NotImplementedError("write your pallas kernel here")


# speedup vs baseline: 2.0428x; 2.0428x over previous
import functools

import jax
import jax.numpy as jnp
from jax import lax
from jax.experimental import pallas as pl
from jax.experimental.pallas import tpu as pltpu
from jax.experimental.pallas import tpu_sc as plsc

N = 10000
E = 320000
H = 128
A = E // 2
NC, NS = 2, 16
NW = NC * NS
CHUNK = 128
W_CHUNKS = 40
A_PAD = NW * W_CHUNKS * CHUNK
T_CHUNKS = A_PAD // (NS * CHUNK)
N_PAD = 10240
ZROWS = 128
TROWS = N_PAD // NS

_mesh = plsc.VectorSubcoreMesh(core_axis_name="c", subcore_axis_name="s")


@functools.partial(
    pl.kernel,
    out_type=jax.ShapeDtypeStruct((A_PAD, H), jnp.float32),
    mesh=_mesh,
    scratch_types=[
        pltpu.VMEM((W_CHUNKS, CHUNK), jnp.int32),
        pltpu.VMEM((W_CHUNKS, CHUNK), jnp.int32),
        pltpu.VMEM((CHUNK, H), jnp.float32),
        pltpu.VMEM((CHUNK, H), jnp.float32),
        pltpu.SemaphoreType.DMA,
    ],
)
def _sc_gather(p0_hbm, p1_hbm, idx_hbm, pre_hbm, idx0_v, idx1_v, g0_v, g1_v, sem):
    c = lax.axis_index("c")
    s = lax.axis_index("s")
    w = s * NC + c
    pltpu.sync_copy(idx_hbm.at[0, pl.ds(w * W_CHUNKS, W_CHUNKS)], idx0_v)
    pltpu.sync_copy(idx_hbm.at[1, pl.ds(w * W_CHUNKS, W_CHUNKS)], idx1_v)

    @pl.loop(0, W_CHUNKS)
    def _chunk(k):
        pltpu.async_copy(p0_hbm.at[idx0_v.at[k]], g0_v, sem).wait()
        pltpu.async_copy(p1_hbm.at[idx1_v.at[k]], g1_v, sem).wait()

        @plsc.parallel_loop(0, CHUNK, unroll=4)
        def _row(r):
            for j in range(H // 16):
                sl = pl.ds(j * 16, 16)
                g0_v[r, sl] = g0_v[r, sl] + g1_v[r, sl]

        pltpu.sync_copy(
            g0_v, pre_hbm.at[pl.ds(w * (W_CHUNKS * CHUNK) + k * CHUNK, CHUNK)]
        )


@functools.partial(
    pl.kernel,
    out_type=jax.ShapeDtypeStruct((2, N_PAD, H), jnp.float32),
    mesh=_mesh,
    scratch_types=[
        pltpu.VMEM((T_CHUNKS, CHUNK), jnp.int32),
        pltpu.VMEM((CHUNK, H), jnp.float32),
        pltpu.VMEM((ZROWS, H), jnp.float32),
        pltpu.VMEM_SHARED((N_PAD, H), jnp.float32),
    ],
)
def _sc_scatter(vals_hbm, idx_hbm, s_hbm, idx_v, vals_v, zero_v, table_sh):
    c = lax.axis_index("c")
    s = lax.axis_index("s")

    zvec = jnp.zeros((16,), jnp.float32)

    @pl.loop(0, ZROWS)
    def _zrow(r):
        for j in range(H // 16):
            zero_v[r, pl.ds(j * 16, 16)] = zvec

    for q in range(TROWS // ZROWS):
        pltpu.sync_copy(zero_v, table_sh.at[pl.ds(s * TROWS + q * ZROWS, ZROWS)])

    plsc.subcore_barrier()

    pltpu.sync_copy(idx_hbm.at[c, pl.ds(s * T_CHUNKS, T_CHUNKS)], idx_v)

    @pl.loop(0, T_CHUNKS)
    def _chunk(k):
        base = s * (T_CHUNKS * CHUNK) + k * CHUNK
        pltpu.sync_copy(vals_hbm.at[c, pl.ds(base, CHUNK)], vals_v)
        pltpu.sync_copy(vals_v, table_sh.at[idx_v.at[k]], add=True)

    plsc.subcore_barrier()

    pltpu.sync_copy(
        table_sh.at[pl.ds(s * TROWS, TROWS)], s_hbm.at[c, pl.ds(s * TROWS, TROWS)]
    )


def _pre_body(h_ref, w1a_ref, w1b_ref, b1_ref, o_ref):
    h = h_ref[...]
    o_ref[0] = jnp.dot(h, w1a_ref[...], preferred_element_type=jnp.float32) + b1_ref[...]
    o_ref[1] = jnp.dot(h, w1b_ref[...], preferred_element_type=jnp.float32)


def _tc_pre(h, w1a, w1b, b1):
    return pl.pallas_call(
        _pre_body,
        out_shape=jax.ShapeDtypeStruct((2, N, H), jnp.float32),
    )(h, w1a, w1b, b1)


RB = 1280


def _atom_body(pre_ref, g_ref, be_ref, w2a_ref, w2b_ref, b2a_ref, b2b_ref, o_ref):
    i = pl.program_id(0)
    x = pre_ref[...]
    m = jnp.mean(x, axis=-1, keepdims=True)
    v = jnp.mean((x - m) * (x - m), axis=-1, keepdims=True)
    y = (x - m) / jnp.sqrt(v + 1e-5) * g_ref[...] + be_ref[...]
    z = jnp.maximum(y, 0.0)
    row = i * RB + lax.broadcasted_iota(jnp.int32, (RB, 1), 0)
    mask = jnp.where(row < A, 1.0, 0.0)
    aL = jnp.dot(z, w2a_ref[...], preferred_element_type=jnp.float32) + b2a_ref[...]
    aR = jnp.dot(z, w2b_ref[...], preferred_element_type=jnp.float32) + b2b_ref[...]
    o_ref[0] = aL * mask
    o_ref[1] = aR * mask


def _tc_atom(pre, g1, be1, w2a, w2b, b2a, b2b):
    nblk = A_PAD // RB
    return pl.pallas_call(
        _atom_body,
        grid=(nblk,),
        in_specs=[
            pl.BlockSpec((RB, H), lambda i: (i, 0)),
            pl.BlockSpec((1, H), lambda i: (0, 0)),
            pl.BlockSpec((1, H), lambda i: (0, 0)),
            pl.BlockSpec((H, H), lambda i: (0, 0)),
            pl.BlockSpec((H, H), lambda i: (0, 0)),
            pl.BlockSpec((1, H), lambda i: (0, 0)),
            pl.BlockSpec((1, H), lambda i: (0, 0)),
        ],
        out_specs=pl.BlockSpec((2, RB, H), lambda i: (0, i, 0)),
        out_shape=jax.ShapeDtypeStruct((2, A_PAD, H), jnp.float32),
    )(pre, g1, be1, w2a, w2b, b2a, b2b)


def _upd_body(h_ref, s_ref, w1a_ref, w1b_ref, b1_ref, g1_ref, be1_ref, w2_ref, b2_ref, o_ref):
    h = h_ref[...]
    st = s_ref[...]
    agg = st[0, :N] + st[1, :N]
    t = (
        jnp.dot(h, w1a_ref[...], preferred_element_type=jnp.float32)
        + jnp.dot(agg, w1b_ref[...], preferred_element_type=jnp.float32)
        + b1_ref[...]
    )
    m = jnp.mean(t, axis=-1, keepdims=True)
    v = jnp.mean((t - m) * (t - m), axis=-1, keepdims=True)
    y = (t - m) / jnp.sqrt(v + 1e-5) * g1_ref[...] + be1_ref[...]
    z = jnp.maximum(y, 0.0)
    o_ref[...] = jnp.dot(z, w2_ref[...], preferred_element_type=jnp.float32) + b2_ref[...]


def _tc_update(h, s, u1a, u1b, ub1, ug1, ube1, uW2, ub2):
    return pl.pallas_call(
        _upd_body,
        out_shape=jax.ShapeDtypeStruct((N, H), jnp.float32),
    )(h, s, u1a, u1b, ub1, ug1, ube1, uW2, ub2)


def _read_body(h_ref, w1_ref, b1_ref, g1_ref, be1_ref, w2_ref, b2_ref, o_ref):
    pooled = jnp.sum(h_ref[...], axis=0, keepdims=True)
    t = jnp.dot(pooled, w1_ref[...], preferred_element_type=jnp.float32) + b1_ref[...]
    m = jnp.mean(t, axis=-1, keepdims=True)
    v = jnp.mean((t - m) * (t - m), axis=-1, keepdims=True)
    y = (t - m) / jnp.sqrt(v + 1e-5) * g1_ref[...] + be1_ref[...]
    sp = jnp.where(y > 20.0, y, jnp.log1p(jnp.exp(jnp.minimum(y, 20.0))))
    z = y * jnp.tanh(sp)
    o_ref[...] = jnp.dot(z, w2_ref[...], preferred_element_type=jnp.float32) + b2_ref[...]


def _tc_readout(h, rW1, rb1, rg1, rbe1, rW2, rb2):
    return pl.pallas_call(
        _read_body,
        out_shape=jax.ShapeDtypeStruct((1, 1), jnp.float32),
    )(h, rW1, rb1, rg1, rbe1, rW2, rb2)


def kernel(x, edge_index, pW1, pb1, pg1, pbe1, pW2, pb2, uW1, ub1, ug1, ube1,
           uW2, ub2, rW1, rb1, rg1, rbe1, rW2, rb2):
    src = edge_index[0]
    a0 = src[0::2]
    a1 = src[1::2]
    pad = jnp.zeros((A_PAD - A,), jnp.int32)
    idx01 = jnp.stack([
        jnp.concatenate([a0, pad]).reshape(NW * W_CHUNKS, CHUNK),
        jnp.concatenate([a1, pad]).reshape(NW * W_CHUNKS, CHUNK),
    ])

    w1a, w1b = pW1[:H], pW1[H:]
    w2a, w2b = pW2[:, :H], pW2[:, H:]
    u1a, u1b = uW1[:H], uW1[H:]
    b1 = pb1.reshape(1, H)
    g1 = pg1.reshape(1, H)
    be1 = pbe1.reshape(1, H)
    b2a = pb2[:H].reshape(1, H)
    b2b = pb2[H:].reshape(1, H)
    ub1_ = ub1.reshape(1, 2 * H)
    ug1_ = ug1.reshape(1, 2 * H)
    ube1_ = ube1.reshape(1, 2 * H)
    ub2_ = ub2.reshape(1, H)
    rb1_ = rb1.reshape(1, 2 * H)
    rg1_ = rg1.reshape(1, 2 * H)
    rbe1_ = rbe1.reshape(1, 2 * H)
    rb2_ = rb2.reshape(1, 1)

    h = x
    for _ in range(3):
        p = _tc_pre(h, w1a, w1b, b1)
        pre = _sc_gather(p[0], p[1], idx01)
        atom = _tc_atom(pre, g1, be1, w2a, w2b, b2a, b2b)
        s = _sc_scatter(atom, idx01)
        h = _tc_update(h, s, u1a, u1b, ub1_, ug1_, ube1_, uW2, ub2_)

    out = _tc_readout(h, rW1, rb1_, rg1_, rbe1_, rW2, rb2_)
    return out.reshape(-1)


# speedup vs baseline: 2.8005x; 1.3709x over previous
import functools

import jax
import jax.numpy as jnp
from jax import lax
from jax.experimental import pallas as pl
from jax.experimental.pallas import tpu as pltpu
from jax.experimental.pallas import tpu_sc as plsc

N = 10000
E = 320000
H = 128
A = E // 2
NC, NS = 2, 16
NW = NC * NS
CHUNK = 128
W_CHUNKS = 40
A_PAD = NW * W_CHUNKS * CHUNK
T_CHUNKS = A_PAD // (NS * CHUNK)
N_PAD = 10240
ZROWS = 128
TROWS = N_PAD // NS

_mesh = plsc.VectorSubcoreMesh(core_axis_name="c", subcore_axis_name="s")


@functools.partial(
    pl.kernel,
    out_type=jax.ShapeDtypeStruct((A_PAD, H), jnp.float32),
    mesh=_mesh,
    scratch_types=[
        pltpu.VMEM((W_CHUNKS, CHUNK), jnp.int32),
        pltpu.VMEM((W_CHUNKS, CHUNK), jnp.int32),
        pltpu.VMEM((2, CHUNK, H), jnp.float32),
        pltpu.VMEM((2, CHUNK, H), jnp.float32),
        pltpu.SemaphoreType.DMA((2,)),
        pltpu.SemaphoreType.DMA((2,)),
    ],
)
def _sc_gather(p0_hbm, p1_hbm, idx_hbm, pre_hbm, idx0_v, idx1_v, g0_v, g1_v,
               gsem, ssem):
    c = lax.axis_index("c")
    s = lax.axis_index("s")
    w = s * NC + c
    base_row = w * (W_CHUNKS * CHUNK)
    pltpu.sync_copy(idx_hbm.at[0, pl.ds(w * W_CHUNKS, W_CHUNKS)], idx0_v)
    pltpu.sync_copy(idx_hbm.at[1, pl.ds(w * W_CHUNKS, W_CHUNKS)], idx1_v)

    def issue_gather(k, b):
        pltpu.async_copy(p0_hbm.at[idx0_v.at[k]], g0_v.at[b], gsem.at[b])
        pltpu.async_copy(p1_hbm.at[idx1_v.at[k]], g1_v.at[b], gsem.at[b])

    def wait_gather(k, b):
        pltpu.make_async_copy(p0_hbm.at[idx0_v.at[k]], g0_v.at[b], gsem.at[b]).wait()
        pltpu.make_async_copy(p1_hbm.at[idx1_v.at[k]], g1_v.at[b], gsem.at[b]).wait()

    def issue_store(k, b):
        pltpu.async_copy(
            g0_v.at[b], pre_hbm.at[pl.ds(base_row + k * CHUNK, CHUNK)], ssem.at[b]
        )

    def wait_store(k, b):
        pltpu.make_async_copy(
            g0_v.at[b], pre_hbm.at[pl.ds(base_row + k * CHUNK, CHUNK)], ssem.at[b]
        ).wait()

    issue_gather(0, 0)

    @pl.loop(0, W_CHUNKS, step=2)
    def _chunk(k0):
        for b in range(2):
            k = k0 + b
            nb = 1 - b

            @pl.when(k >= 1)
            def _():
                wait_store(k - 1, nb)

            @pl.when(k + 1 < W_CHUNKS)
            def _():
                issue_gather(k + 1, nb)

            wait_gather(k, b)

            @plsc.parallel_loop(0, CHUNK, unroll=4)
            def _row(r):
                for j in range(H // 16):
                    sl = pl.ds(j * 16, 16)
                    g0_v[b, r, sl] = g0_v[b, r, sl] + g1_v[b, r, sl]

            issue_store(k, b)

    wait_store(W_CHUNKS - 1, 1)


@functools.partial(
    pl.kernel,
    out_type=jax.ShapeDtypeStruct((2, N_PAD, H), jnp.float32),
    mesh=_mesh,
    scratch_types=[
        pltpu.VMEM((T_CHUNKS, CHUNK), jnp.int32),
        pltpu.VMEM((CHUNK, H), jnp.float32),
        pltpu.VMEM((ZROWS, H), jnp.float32),
        pltpu.VMEM_SHARED((N_PAD, H), jnp.float32),
    ],
)
def _sc_scatter(vals_hbm, idx_hbm, s_hbm, idx_v, vals_v, zero_v, table_sh):
    c = lax.axis_index("c")
    s = lax.axis_index("s")

    zvec = jnp.zeros((16,), jnp.float32)

    @pl.loop(0, ZROWS)
    def _zrow(r):
        for j in range(H // 16):
            zero_v[r, pl.ds(j * 16, 16)] = zvec

    for q in range(TROWS // ZROWS):
        pltpu.sync_copy(zero_v, table_sh.at[pl.ds(s * TROWS + q * ZROWS, ZROWS)])

    plsc.subcore_barrier()

    pltpu.sync_copy(idx_hbm.at[c, pl.ds(s * T_CHUNKS, T_CHUNKS)], idx_v)

    tile_base = s * (T_CHUNKS * CHUNK)

    @pl.loop(0, T_CHUNKS)
    def _chunk(k):
        pltpu.sync_copy(vals_hbm.at[c, pl.ds(tile_base + k * CHUNK, CHUNK)],
                        vals_v)
        pltpu.sync_copy(vals_v, table_sh.at[idx_v.at[k]], add=True)

    plsc.subcore_barrier()

    pltpu.sync_copy(
        table_sh.at[pl.ds(s * TROWS, TROWS)], s_hbm.at[c, pl.ds(s * TROWS, TROWS)]
    )


def _pre_body(h_ref, w1a_ref, w1b_ref, b1_ref, o_ref):
    h = h_ref[...]
    o_ref[0] = jnp.dot(h, w1a_ref[...], preferred_element_type=jnp.float32) + b1_ref[...]
    o_ref[1] = jnp.dot(h, w1b_ref[...], preferred_element_type=jnp.float32)


def _tc_pre(h, w1a, w1b, b1):
    return pl.pallas_call(
        _pre_body,
        out_shape=jax.ShapeDtypeStruct((2, N, H), jnp.float32),
    )(h, w1a, w1b, b1)


RB = 1280


def _atom_body(pre_ref, g_ref, be_ref, w2a_ref, w2b_ref, b2a_ref, b2b_ref, o_ref):
    i = pl.program_id(0)
    x = pre_ref[...]
    m = jnp.mean(x, axis=-1, keepdims=True)
    v = jnp.mean((x - m) * (x - m), axis=-1, keepdims=True)
    y = (x - m) / jnp.sqrt(v + 1e-5) * g_ref[...] + be_ref[...]
    z = jnp.maximum(y, 0.0)
    row = i * RB + lax.broadcasted_iota(jnp.int32, (RB, 1), 0)
    mask = jnp.where(row < A, 1.0, 0.0)
    aL = jnp.dot(z, w2a_ref[...], preferred_element_type=jnp.float32) + b2a_ref[...]
    aR = jnp.dot(z, w2b_ref[...], preferred_element_type=jnp.float32) + b2b_ref[...]
    o_ref[0] = aL * mask
    o_ref[1] = aR * mask


def _tc_atom(pre, g1, be1, w2a, w2b, b2a, b2b):
    nblk = A_PAD // RB
    return pl.pallas_call(
        _atom_body,
        grid=(nblk,),
        in_specs=[
            pl.BlockSpec((RB, H), lambda i: (i, 0)),
            pl.BlockSpec((1, H), lambda i: (0, 0)),
            pl.BlockSpec((1, H), lambda i: (0, 0)),
            pl.BlockSpec((H, H), lambda i: (0, 0)),
            pl.BlockSpec((H, H), lambda i: (0, 0)),
            pl.BlockSpec((1, H), lambda i: (0, 0)),
            pl.BlockSpec((1, H), lambda i: (0, 0)),
        ],
        out_specs=pl.BlockSpec((2, RB, H), lambda i: (0, i, 0)),
        out_shape=jax.ShapeDtypeStruct((2, A_PAD, H), jnp.float32),
    )(pre, g1, be1, w2a, w2b, b2a, b2b)


def _upd_body(h_ref, s_ref, w1a_ref, w1b_ref, b1_ref, g1_ref, be1_ref, w2_ref, b2_ref, o_ref):
    h = h_ref[...]
    st = s_ref[...]
    agg = st[0, :N] + st[1, :N]
    t = (
        jnp.dot(h, w1a_ref[...], preferred_element_type=jnp.float32)
        + jnp.dot(agg, w1b_ref[...], preferred_element_type=jnp.float32)
        + b1_ref[...]
    )
    m = jnp.mean(t, axis=-1, keepdims=True)
    v = jnp.mean((t - m) * (t - m), axis=-1, keepdims=True)
    y = (t - m) / jnp.sqrt(v + 1e-5) * g1_ref[...] + be1_ref[...]
    z = jnp.maximum(y, 0.0)
    o_ref[...] = jnp.dot(z, w2_ref[...], preferred_element_type=jnp.float32) + b2_ref[...]


def _tc_update(h, s, u1a, u1b, ub1, ug1, ube1, uW2, ub2):
    return pl.pallas_call(
        _upd_body,
        out_shape=jax.ShapeDtypeStruct((N, H), jnp.float32),
    )(h, s, u1a, u1b, ub1, ug1, ube1, uW2, ub2)


def _read_body(h_ref, w1_ref, b1_ref, g1_ref, be1_ref, w2_ref, b2_ref, o_ref):
    pooled = jnp.sum(h_ref[...], axis=0, keepdims=True)
    t = jnp.dot(pooled, w1_ref[...], preferred_element_type=jnp.float32) + b1_ref[...]
    m = jnp.mean(t, axis=-1, keepdims=True)
    v = jnp.mean((t - m) * (t - m), axis=-1, keepdims=True)
    y = (t - m) / jnp.sqrt(v + 1e-5) * g1_ref[...] + be1_ref[...]
    sp = jnp.where(y > 20.0, y, jnp.log1p(jnp.exp(jnp.minimum(y, 20.0))))
    z = y * jnp.tanh(sp)
    o_ref[...] = jnp.dot(z, w2_ref[...], preferred_element_type=jnp.float32) + b2_ref[...]


def _tc_readout(h, rW1, rb1, rg1, rbe1, rW2, rb2):
    return pl.pallas_call(
        _read_body,
        out_shape=jax.ShapeDtypeStruct((1, 1), jnp.float32),
    )(h, rW1, rb1, rg1, rbe1, rW2, rb2)


def kernel(x, edge_index, pW1, pb1, pg1, pbe1, pW2, pb2, uW1, ub1, ug1, ube1,
           uW2, ub2, rW1, rb1, rg1, rbe1, rW2, rb2):
    src = edge_index[0]
    a0 = src[0::2]
    a1 = src[1::2]
    pad = jnp.zeros((A_PAD - A,), jnp.int32)
    idx01 = jnp.stack([
        jnp.concatenate([a0, pad]).reshape(NW * W_CHUNKS, CHUNK),
        jnp.concatenate([a1, pad]).reshape(NW * W_CHUNKS, CHUNK),
    ])

    w1a, w1b = pW1[:H], pW1[H:]
    w2a, w2b = pW2[:, :H], pW2[:, H:]
    u1a, u1b = uW1[:H], uW1[H:]
    b1 = pb1.reshape(1, H)
    g1 = pg1.reshape(1, H)
    be1 = pbe1.reshape(1, H)
    b2a = pb2[:H].reshape(1, H)
    b2b = pb2[H:].reshape(1, H)
    ub1_ = ub1.reshape(1, 2 * H)
    ug1_ = ug1.reshape(1, 2 * H)
    ube1_ = ube1.reshape(1, 2 * H)
    ub2_ = ub2.reshape(1, H)
    rb1_ = rb1.reshape(1, 2 * H)
    rg1_ = rg1.reshape(1, 2 * H)
    rbe1_ = rbe1.reshape(1, 2 * H)
    rb2_ = rb2.reshape(1, 1)

    h = x
    for _ in range(3):
        p = _tc_pre(h, w1a, w1b, b1)
        pre = _sc_gather(p[0], p[1], idx01)
        atom = _tc_atom(pre, g1, be1, w2a, w2b, b2a, b2b)
        s = _sc_scatter(atom, idx01)
        h = _tc_update(h, s, u1a, u1b, ub1_, ug1_, ube1_, uW2, ub2_)

    out = _tc_readout(h, rW1, rb1_, rg1_, rbe1_, rW2, rb2_)
    return out.reshape(-1)


# speedup vs baseline: 3.0952x; 1.1052x over previous
import functools

import jax
import jax.numpy as jnp
from jax import lax
from jax.experimental import pallas as pl
from jax.experimental.pallas import tpu as pltpu
from jax.experimental.pallas import tpu_sc as plsc

N = 10000
E = 320000
H = 128
A = E // 2
NC, NS = 2, 16
NW = NC * NS
CHUNK = 128
W_CHUNKS = 40
A_PAD = NW * W_CHUNKS * CHUNK
T_CHUNKS = A_PAD // (NS * CHUNK)
N_PAD = 10240
ZROWS = 16
TROWS = N_PAD // NS

_mesh = plsc.VectorSubcoreMesh(core_axis_name="c", subcore_axis_name="s")


@functools.partial(
    pl.kernel,
    out_type=jax.ShapeDtypeStruct((A_PAD, H), jnp.float32),
    mesh=_mesh,
    scratch_types=[
        pltpu.VMEM((W_CHUNKS, CHUNK), jnp.int32),
        pltpu.VMEM((W_CHUNKS, CHUNK), jnp.int32),
        pltpu.VMEM((2, CHUNK, H), jnp.float32),
        pltpu.VMEM((2, CHUNK, H), jnp.float32),
        pltpu.SemaphoreType.DMA((2,)),
        pltpu.SemaphoreType.DMA((2,)),
    ],
)
def _sc_gather(p0_hbm, p1_hbm, idx_hbm, pre_hbm, idx0_v, idx1_v, g0_v, g1_v,
               gsem, ssem):
    c = lax.axis_index("c")
    s = lax.axis_index("s")
    w = s * NC + c
    base_row = w * (W_CHUNKS * CHUNK)
    pltpu.sync_copy(idx_hbm.at[0, pl.ds(w * W_CHUNKS, W_CHUNKS)], idx0_v)
    pltpu.sync_copy(idx_hbm.at[1, pl.ds(w * W_CHUNKS, W_CHUNKS)], idx1_v)

    def issue_gather(k, b):
        pltpu.async_copy(p0_hbm.at[idx0_v.at[k]], g0_v.at[b], gsem.at[b])
        pltpu.async_copy(p1_hbm.at[idx1_v.at[k]], g1_v.at[b], gsem.at[b])

    def wait_gather(k, b):
        pltpu.make_async_copy(p0_hbm.at[idx0_v.at[k]], g0_v.at[b], gsem.at[b]).wait()
        pltpu.make_async_copy(p1_hbm.at[idx1_v.at[k]], g1_v.at[b], gsem.at[b]).wait()

    def issue_store(k, b):
        pltpu.async_copy(
            g0_v.at[b], pre_hbm.at[pl.ds(base_row + k * CHUNK, CHUNK)], ssem.at[b]
        )

    def wait_store(k, b):
        pltpu.make_async_copy(
            g0_v.at[b], pre_hbm.at[pl.ds(base_row + k * CHUNK, CHUNK)], ssem.at[b]
        ).wait()

    issue_gather(0, 0)

    @pl.loop(0, W_CHUNKS, step=2)
    def _chunk(k0):
        for b in range(2):
            k = k0 + b
            nb = 1 - b

            @pl.when(k >= 1)
            def _():
                wait_store(k - 1, nb)

            @pl.when(k + 1 < W_CHUNKS)
            def _():
                issue_gather(k + 1, nb)

            wait_gather(k, b)

            @plsc.parallel_loop(0, CHUNK, unroll=4)
            def _row(r):
                for j in range(H // 16):
                    sl = pl.ds(j * 16, 16)
                    g0_v[b, r, sl] = g0_v[b, r, sl] + g1_v[b, r, sl]

            issue_store(k, b)

    wait_store(W_CHUNKS - 1, 1)


@functools.partial(
    pl.kernel,
    out_type=jax.ShapeDtypeStruct((2, N_PAD, H), jnp.float32),
    mesh=_mesh,
    scratch_types=[
        pltpu.VMEM((T_CHUNKS, CHUNK), jnp.int32),
        pltpu.VMEM((CHUNK, H), jnp.float32),
        pltpu.VMEM((CHUNK, H), jnp.float32),
        pltpu.VMEM((ZROWS, H), jnp.float32),
        pltpu.VMEM_SHARED((N_PAD, H), jnp.float32),
        pltpu.SemaphoreType.DMA,
        pltpu.SemaphoreType.DMA,
    ],
)
def _sc_scatter(vals_hbm, idx_hbm, s_hbm, idx_v, vals_a, vals_b, zero_v,
                table_sh, lsem_a, lsem_b):
    c = lax.axis_index("c")
    s = lax.axis_index("s")

    zvec = jnp.zeros((16,), jnp.float32)

    @pl.loop(0, ZROWS)
    def _zrow(r):
        for j in range(H // 16):
            zero_v[r, pl.ds(j * 16, 16)] = zvec

    for q in range(TROWS // ZROWS):
        pltpu.sync_copy(zero_v, table_sh.at[pl.ds(s * TROWS + q * ZROWS, ZROWS)])

    plsc.subcore_barrier()

    pltpu.sync_copy(idx_hbm.at[c, pl.ds(s * T_CHUNKS, T_CHUNKS)], idx_v)

    tile_base = s * (T_CHUNKS * CHUNK)

    def issue_load(k, ref, sem):
        pltpu.async_copy(
            vals_hbm.at[c, pl.ds(tile_base + k * CHUNK, CHUNK)], ref, sem
        )

    def wait_load(k, ref, sem):
        pltpu.make_async_copy(
            vals_hbm.at[c, pl.ds(tile_base + k * CHUNK, CHUNK)], ref, sem
        ).wait()

    issue_load(0, vals_a, lsem_a)

    @pl.loop(0, T_CHUNKS, step=2)
    def _chunk(k0):
        issue_load(k0 + 1, vals_b, lsem_b)
        wait_load(k0, vals_a, lsem_a)
        pltpu.sync_copy(vals_a, table_sh.at[idx_v.at[k0]], add=True)

        @pl.when(k0 + 2 < T_CHUNKS)
        def _():
            issue_load(k0 + 2, vals_a, lsem_a)

        wait_load(k0 + 1, vals_b, lsem_b)
        pltpu.sync_copy(vals_b, table_sh.at[idx_v.at[k0 + 1]], add=True)

    plsc.subcore_barrier()

    pltpu.sync_copy(
        table_sh.at[pl.ds(s * TROWS, TROWS)], s_hbm.at[c, pl.ds(s * TROWS, TROWS)]
    )


def _pre_body(h_ref, w1a_ref, w1b_ref, b1_ref, o_ref):
    h = h_ref[...]
    o_ref[0] = jnp.dot(h, w1a_ref[...], preferred_element_type=jnp.float32) + b1_ref[...]
    o_ref[1] = jnp.dot(h, w1b_ref[...], preferred_element_type=jnp.float32)


def _tc_pre(h, w1a, w1b, b1):
    return pl.pallas_call(
        _pre_body,
        out_shape=jax.ShapeDtypeStruct((2, N, H), jnp.float32),
    )(h, w1a, w1b, b1)


RB = 1280


def _atom_body(pre_ref, g_ref, be_ref, w2a_ref, w2b_ref, b2a_ref, b2b_ref, o_ref):
    i = pl.program_id(0)
    x = pre_ref[...]
    m = jnp.mean(x, axis=-1, keepdims=True)
    v = jnp.mean((x - m) * (x - m), axis=-1, keepdims=True)
    y = (x - m) / jnp.sqrt(v + 1e-5) * g_ref[...] + be_ref[...]
    z = jnp.maximum(y, 0.0)
    row = i * RB + lax.broadcasted_iota(jnp.int32, (RB, 1), 0)
    mask = jnp.where(row < A, 1.0, 0.0)
    aL = jnp.dot(z, w2a_ref[...], preferred_element_type=jnp.float32) + b2a_ref[...]
    aR = jnp.dot(z, w2b_ref[...], preferred_element_type=jnp.float32) + b2b_ref[...]
    o_ref[0] = aL * mask
    o_ref[1] = aR * mask


def _tc_atom(pre, g1, be1, w2a, w2b, b2a, b2b):
    nblk = A_PAD // RB
    return pl.pallas_call(
        _atom_body,
        grid=(nblk,),
        in_specs=[
            pl.BlockSpec((RB, H), lambda i: (i, 0)),
            pl.BlockSpec((1, H), lambda i: (0, 0)),
            pl.BlockSpec((1, H), lambda i: (0, 0)),
            pl.BlockSpec((H, H), lambda i: (0, 0)),
            pl.BlockSpec((H, H), lambda i: (0, 0)),
            pl.BlockSpec((1, H), lambda i: (0, 0)),
            pl.BlockSpec((1, H), lambda i: (0, 0)),
        ],
        out_specs=pl.BlockSpec((2, RB, H), lambda i: (0, i, 0)),
        out_shape=jax.ShapeDtypeStruct((2, A_PAD, H), jnp.float32),
    )(pre, g1, be1, w2a, w2b, b2a, b2b)


def _upd_body(h_ref, s_ref, w1a_ref, w1b_ref, b1_ref, g1_ref, be1_ref, w2_ref, b2_ref, o_ref):
    h = h_ref[...]
    st = s_ref[...]
    agg = st[0, :N] + st[1, :N]
    t = (
        jnp.dot(h, w1a_ref[...], preferred_element_type=jnp.float32)
        + jnp.dot(agg, w1b_ref[...], preferred_element_type=jnp.float32)
        + b1_ref[...]
    )
    m = jnp.mean(t, axis=-1, keepdims=True)
    v = jnp.mean((t - m) * (t - m), axis=-1, keepdims=True)
    y = (t - m) / jnp.sqrt(v + 1e-5) * g1_ref[...] + be1_ref[...]
    z = jnp.maximum(y, 0.0)
    o_ref[...] = jnp.dot(z, w2_ref[...], preferred_element_type=jnp.float32) + b2_ref[...]


def _tc_update(h, s, u1a, u1b, ub1, ug1, ube1, uW2, ub2):
    return pl.pallas_call(
        _upd_body,
        out_shape=jax.ShapeDtypeStruct((N, H), jnp.float32),
    )(h, s, u1a, u1b, ub1, ug1, ube1, uW2, ub2)


def _read_body(h_ref, w1_ref, b1_ref, g1_ref, be1_ref, w2_ref, b2_ref, o_ref):
    pooled = jnp.sum(h_ref[...], axis=0, keepdims=True)
    t = jnp.dot(pooled, w1_ref[...], preferred_element_type=jnp.float32) + b1_ref[...]
    m = jnp.mean(t, axis=-1, keepdims=True)
    v = jnp.mean((t - m) * (t - m), axis=-1, keepdims=True)
    y = (t - m) / jnp.sqrt(v + 1e-5) * g1_ref[...] + be1_ref[...]
    sp = jnp.where(y > 20.0, y, jnp.log1p(jnp.exp(jnp.minimum(y, 20.0))))
    z = y * jnp.tanh(sp)
    o_ref[...] = jnp.dot(z, w2_ref[...], preferred_element_type=jnp.float32) + b2_ref[...]


def _tc_readout(h, rW1, rb1, rg1, rbe1, rW2, rb2):
    return pl.pallas_call(
        _read_body,
        out_shape=jax.ShapeDtypeStruct((1, 1), jnp.float32),
    )(h, rW1, rb1, rg1, rbe1, rW2, rb2)


def kernel(x, edge_index, pW1, pb1, pg1, pbe1, pW2, pb2, uW1, ub1, ug1, ube1,
           uW2, ub2, rW1, rb1, rg1, rbe1, rW2, rb2):
    src = edge_index[0]
    a0 = src[0::2]
    a1 = src[1::2]
    pad = jnp.zeros((A_PAD - A,), jnp.int32)
    idx01 = jnp.stack([
        jnp.concatenate([a0, pad]).reshape(NW * W_CHUNKS, CHUNK),
        jnp.concatenate([a1, pad]).reshape(NW * W_CHUNKS, CHUNK),
    ])

    w1a, w1b = pW1[:H], pW1[H:]
    w2a, w2b = pW2[:, :H], pW2[:, H:]
    u1a, u1b = uW1[:H], uW1[H:]
    b1 = pb1.reshape(1, H)
    g1 = pg1.reshape(1, H)
    be1 = pbe1.reshape(1, H)
    b2a = pb2[:H].reshape(1, H)
    b2b = pb2[H:].reshape(1, H)
    ub1_ = ub1.reshape(1, 2 * H)
    ug1_ = ug1.reshape(1, 2 * H)
    ube1_ = ube1.reshape(1, 2 * H)
    ub2_ = ub2.reshape(1, H)
    rb1_ = rb1.reshape(1, 2 * H)
    rg1_ = rg1.reshape(1, 2 * H)
    rbe1_ = rbe1.reshape(1, 2 * H)
    rb2_ = rb2.reshape(1, 1)

    h = x
    for _ in range(3):
        p = _tc_pre(h, w1a, w1b, b1)
        pre = _sc_gather(p[0], p[1], idx01)
        atom = _tc_atom(pre, g1, be1, w2a, w2b, b2a, b2b)
        s = _sc_scatter(atom, idx01)
        h = _tc_update(h, s, u1a, u1b, ub1_, ug1_, ube1_, uW2, ub2_)

    out = _tc_readout(h, rW1, rb1_, rg1_, rbe1_, rW2, rb2_)
    return out.reshape(-1)


# speedup vs baseline: 3.1092x; 1.0045x over previous
import functools

import jax
import jax.numpy as jnp
from jax import lax
from jax.experimental import pallas as pl
from jax.experimental.pallas import tpu as pltpu
from jax.experimental.pallas import tpu_sc as plsc

N = 10000
E = 320000
H = 128
A = E // 2
NC, NS = 2, 16
NW = NC * NS
CHUNK = 128
W_CHUNKS = 40
A_PAD = NW * W_CHUNKS * CHUNK
T_CHUNKS = A_PAD // (NS * CHUNK)
N_PAD = 10240
ZROWS = 16
TROWS = N_PAD // NS

_mesh = plsc.VectorSubcoreMesh(core_axis_name="c", subcore_axis_name="s")


@functools.partial(
    pl.kernel,
    out_type=jax.ShapeDtypeStruct((A_PAD, H), jnp.float32),
    mesh=_mesh,
    scratch_types=[
        pltpu.VMEM((W_CHUNKS, CHUNK), jnp.int32),
        pltpu.VMEM((W_CHUNKS, CHUNK), jnp.int32),
        pltpu.VMEM((3, CHUNK, H), jnp.float32),
        pltpu.VMEM((3, CHUNK, H), jnp.float32),
        pltpu.SemaphoreType.DMA((3,)),
        pltpu.SemaphoreType.DMA((3,)),
    ],
)
def _sc_gather(p0_hbm, p1_hbm, idx_hbm, pre_hbm, idx0_v, idx1_v, g0_v, g1_v,
               gsem, ssem):
    c = lax.axis_index("c")
    s = lax.axis_index("s")
    w = s * NC + c
    base_row = w * (W_CHUNKS * CHUNK)
    pltpu.sync_copy(idx_hbm.at[0, pl.ds(w * W_CHUNKS, W_CHUNKS)], idx0_v)
    pltpu.sync_copy(idx_hbm.at[1, pl.ds(w * W_CHUNKS, W_CHUNKS)], idx1_v)

    def issue_gather(k, b):
        pltpu.async_copy(p0_hbm.at[idx0_v.at[k]], g0_v.at[b], gsem.at[b])
        pltpu.async_copy(p1_hbm.at[idx1_v.at[k]], g1_v.at[b], gsem.at[b])

    def wait_gather(k, b):
        pltpu.make_async_copy(p0_hbm.at[idx0_v.at[k]], g0_v.at[b], gsem.at[b]).wait()
        pltpu.make_async_copy(p1_hbm.at[idx1_v.at[k]], g1_v.at[b], gsem.at[b]).wait()

    def issue_store(k, b):
        pltpu.async_copy(
            g0_v.at[b], pre_hbm.at[pl.ds(base_row + k * CHUNK, CHUNK)], ssem.at[b]
        )

    def wait_store(k, b):
        pltpu.make_async_copy(
            g0_v.at[b], pre_hbm.at[pl.ds(base_row + k * CHUNK, CHUNK)], ssem.at[b]
        ).wait()

    def compute_add(b):
        @plsc.parallel_loop(0, CHUNK, unroll=4)
        def _row(r):
            for jj in range(H // 16):
                sl = pl.ds(jj * 16, 16)
                g0_v[b, r, sl] = g0_v[b, r, sl] + g1_v[b, r, sl]

    issue_gather(0, 0)
    issue_gather(1, 1)

    @pl.loop(0, W_CHUNKS - 1, step=3)
    def _chunk(j0):
        for d in range(3):
            j = j0 + d
            b = d
            nb = (d + 2) % 3

            @pl.when(j + 2 < W_CHUNKS)
            def _():
                @pl.when(j >= 1)
                def _():
                    wait_store(j - 1, nb)

                issue_gather(j + 2, nb)

            wait_gather(j, b)
            compute_add(b)
            issue_store(j, b)

    wait_gather(W_CHUNKS - 1, (W_CHUNKS - 1) % 3)
    compute_add((W_CHUNKS - 1) % 3)
    issue_store(W_CHUNKS - 1, (W_CHUNKS - 1) % 3)

    wait_store(W_CHUNKS - 3, (W_CHUNKS - 3) % 3)
    wait_store(W_CHUNKS - 2, (W_CHUNKS - 2) % 3)
    wait_store(W_CHUNKS - 1, (W_CHUNKS - 1) % 3)


@functools.partial(
    pl.kernel,
    out_type=jax.ShapeDtypeStruct((2, N_PAD, H), jnp.float32),
    mesh=_mesh,
    scratch_types=[
        pltpu.VMEM((T_CHUNKS, CHUNK), jnp.int32),
        pltpu.VMEM((CHUNK, H), jnp.float32),
        pltpu.VMEM((CHUNK, H), jnp.float32),
        pltpu.VMEM((ZROWS, H), jnp.float32),
        pltpu.VMEM_SHARED((N_PAD, H), jnp.float32),
        pltpu.SemaphoreType.DMA,
        pltpu.SemaphoreType.DMA,
    ],
)
def _sc_scatter(vals_hbm, idx_hbm, s_hbm, idx_v, vals_a, vals_b, zero_v,
                table_sh, lsem_a, lsem_b):
    c = lax.axis_index("c")
    s = lax.axis_index("s")

    zvec = jnp.zeros((16,), jnp.float32)

    @pl.loop(0, ZROWS)
    def _zrow(r):
        for j in range(H // 16):
            zero_v[r, pl.ds(j * 16, 16)] = zvec

    for q in range(TROWS // ZROWS):
        pltpu.sync_copy(zero_v, table_sh.at[pl.ds(s * TROWS + q * ZROWS, ZROWS)])

    plsc.subcore_barrier()

    pltpu.sync_copy(idx_hbm.at[c, pl.ds(s * T_CHUNKS, T_CHUNKS)], idx_v)

    tile_base = s * (T_CHUNKS * CHUNK)

    def issue_load(k, ref, sem):
        pltpu.async_copy(
            vals_hbm.at[c, pl.ds(tile_base + k * CHUNK, CHUNK)], ref, sem
        )

    def wait_load(k, ref, sem):
        pltpu.make_async_copy(
            vals_hbm.at[c, pl.ds(tile_base + k * CHUNK, CHUNK)], ref, sem
        ).wait()

    issue_load(0, vals_a, lsem_a)

    @pl.loop(0, T_CHUNKS, step=2)
    def _chunk(k0):
        issue_load(k0 + 1, vals_b, lsem_b)
        wait_load(k0, vals_a, lsem_a)
        pltpu.sync_copy(vals_a, table_sh.at[idx_v.at[k0]], add=True)

        @pl.when(k0 + 2 < T_CHUNKS)
        def _():
            issue_load(k0 + 2, vals_a, lsem_a)

        wait_load(k0 + 1, vals_b, lsem_b)
        pltpu.sync_copy(vals_b, table_sh.at[idx_v.at[k0 + 1]], add=True)

    plsc.subcore_barrier()

    pltpu.sync_copy(
        table_sh.at[pl.ds(s * TROWS, TROWS)], s_hbm.at[c, pl.ds(s * TROWS, TROWS)]
    )


def _pre_body(h_ref, w1a_ref, w1b_ref, b1_ref, o_ref):
    h = h_ref[...]
    o_ref[0] = jnp.dot(h, w1a_ref[...], preferred_element_type=jnp.float32) + b1_ref[...]
    o_ref[1] = jnp.dot(h, w1b_ref[...], preferred_element_type=jnp.float32)


def _tc_pre(h, w1a, w1b, b1):
    return pl.pallas_call(
        _pre_body,
        out_shape=jax.ShapeDtypeStruct((2, N, H), jnp.float32),
    )(h, w1a, w1b, b1)


RB = 1280


def _atom_body(pre_ref, g_ref, be_ref, w2a_ref, w2b_ref, b2a_ref, b2b_ref, o_ref):
    i = pl.program_id(0)
    x = pre_ref[...]
    m = jnp.mean(x, axis=-1, keepdims=True)
    v = jnp.mean((x - m) * (x - m), axis=-1, keepdims=True)
    y = (x - m) / jnp.sqrt(v + 1e-5) * g_ref[...] + be_ref[...]
    z = jnp.maximum(y, 0.0)
    row = i * RB + lax.broadcasted_iota(jnp.int32, (RB, 1), 0)
    mask = jnp.where(row < A, 1.0, 0.0)
    aL = jnp.dot(z, w2a_ref[...], preferred_element_type=jnp.float32) + b2a_ref[...]
    aR = jnp.dot(z, w2b_ref[...], preferred_element_type=jnp.float32) + b2b_ref[...]
    o_ref[0] = aL * mask
    o_ref[1] = aR * mask


def _tc_atom(pre, g1, be1, w2a, w2b, b2a, b2b):
    nblk = A_PAD // RB
    return pl.pallas_call(
        _atom_body,
        grid=(nblk,),
        in_specs=[
            pl.BlockSpec((RB, H), lambda i: (i, 0)),
            pl.BlockSpec((1, H), lambda i: (0, 0)),
            pl.BlockSpec((1, H), lambda i: (0, 0)),
            pl.BlockSpec((H, H), lambda i: (0, 0)),
            pl.BlockSpec((H, H), lambda i: (0, 0)),
            pl.BlockSpec((1, H), lambda i: (0, 0)),
            pl.BlockSpec((1, H), lambda i: (0, 0)),
        ],
        out_specs=pl.BlockSpec((2, RB, H), lambda i: (0, i, 0)),
        out_shape=jax.ShapeDtypeStruct((2, A_PAD, H), jnp.float32),
    )(pre, g1, be1, w2a, w2b, b2a, b2b)


def _upd_body(h_ref, s_ref, w1a_ref, w1b_ref, b1_ref, g1_ref, be1_ref, w2_ref, b2_ref, o_ref):
    h = h_ref[...]
    st = s_ref[...]
    agg = st[0, :N] + st[1, :N]
    t = (
        jnp.dot(h, w1a_ref[...], preferred_element_type=jnp.float32)
        + jnp.dot(agg, w1b_ref[...], preferred_element_type=jnp.float32)
        + b1_ref[...]
    )
    m = jnp.mean(t, axis=-1, keepdims=True)
    v = jnp.mean((t - m) * (t - m), axis=-1, keepdims=True)
    y = (t - m) / jnp.sqrt(v + 1e-5) * g1_ref[...] + be1_ref[...]
    z = jnp.maximum(y, 0.0)
    o_ref[...] = jnp.dot(z, w2_ref[...], preferred_element_type=jnp.float32) + b2_ref[...]


def _tc_update(h, s, u1a, u1b, ub1, ug1, ube1, uW2, ub2):
    return pl.pallas_call(
        _upd_body,
        out_shape=jax.ShapeDtypeStruct((N, H), jnp.float32),
    )(h, s, u1a, u1b, ub1, ug1, ube1, uW2, ub2)


def _read_body(h_ref, w1_ref, b1_ref, g1_ref, be1_ref, w2_ref, b2_ref, o_ref):
    pooled = jnp.sum(h_ref[...], axis=0, keepdims=True)
    t = jnp.dot(pooled, w1_ref[...], preferred_element_type=jnp.float32) + b1_ref[...]
    m = jnp.mean(t, axis=-1, keepdims=True)
    v = jnp.mean((t - m) * (t - m), axis=-1, keepdims=True)
    y = (t - m) / jnp.sqrt(v + 1e-5) * g1_ref[...] + be1_ref[...]
    sp = jnp.where(y > 20.0, y, jnp.log1p(jnp.exp(jnp.minimum(y, 20.0))))
    z = y * jnp.tanh(sp)
    o_ref[...] = jnp.dot(z, w2_ref[...], preferred_element_type=jnp.float32) + b2_ref[...]


def _tc_readout(h, rW1, rb1, rg1, rbe1, rW2, rb2):
    return pl.pallas_call(
        _read_body,
        out_shape=jax.ShapeDtypeStruct((1, 1), jnp.float32),
    )(h, rW1, rb1, rg1, rbe1, rW2, rb2)


def kernel(x, edge_index, pW1, pb1, pg1, pbe1, pW2, pb2, uW1, ub1, ug1, ube1,
           uW2, ub2, rW1, rb1, rg1, rbe1, rW2, rb2):
    src = edge_index[0]
    a0 = src[0::2]
    a1 = src[1::2]
    pad = jnp.zeros((A_PAD - A,), jnp.int32)
    idx01 = jnp.stack([
        jnp.concatenate([a0, pad]).reshape(NW * W_CHUNKS, CHUNK),
        jnp.concatenate([a1, pad]).reshape(NW * W_CHUNKS, CHUNK),
    ])

    w1a, w1b = pW1[:H], pW1[H:]
    w2a, w2b = pW2[:, :H], pW2[:, H:]
    u1a, u1b = uW1[:H], uW1[H:]
    b1 = pb1.reshape(1, H)
    g1 = pg1.reshape(1, H)
    be1 = pbe1.reshape(1, H)
    b2a = pb2[:H].reshape(1, H)
    b2b = pb2[H:].reshape(1, H)
    ub1_ = ub1.reshape(1, 2 * H)
    ug1_ = ug1.reshape(1, 2 * H)
    ube1_ = ube1.reshape(1, 2 * H)
    ub2_ = ub2.reshape(1, H)
    rb1_ = rb1.reshape(1, 2 * H)
    rg1_ = rg1.reshape(1, 2 * H)
    rbe1_ = rbe1.reshape(1, 2 * H)
    rb2_ = rb2.reshape(1, 1)

    h = x
    for _ in range(3):
        p = _tc_pre(h, w1a, w1b, b1)
        pre = _sc_gather(p[0], p[1], idx01)
        atom = _tc_atom(pre, g1, be1, w2a, w2b, b2a, b2b)
        s = _sc_scatter(atom, idx01)
        h = _tc_update(h, s, u1a, u1b, ub1_, ug1_, ube1_, uW2, ub2_)

    out = _tc_readout(h, rW1, rb1_, rg1_, rbe1_, rW2, rb2_)
    return out.reshape(-1)
